# baseline (speedup 1.0000x reference)
import functools

import numpy as np
import jax
import jax.numpy as jnp
from jax import lax
from jax.experimental import pallas as pl
from jax.experimental.pallas import tpu as pltpu
from jax.experimental.pallas import tpu_sc as plsc

_LAYOUT = [3, 128, 256, 256, 128]
_NOFF = np.cumsum([0] + _LAYOUT)
_ROWS = [_LAYOUT[j] * _LAYOUT[j + 1] for j in range(4)]
_EOFF = np.cumsum([0] + _ROWS)
_CROP_F = [6, 7, 8, 11, 12, 13, 16, 17, 18]
_B = 16
_EFLAT = int(_EOFF[4]) * 25
_CPC = 12800
_WN = [384 * 9, 32768 * 9, 65536 * 9, 32768 * 25]
_BN = [128, 256, 256, 128]
_FR = (np.float32(0.75), np.float32(0.5), np.float32(0.25))


def _unit_rows():
    def row(src, base, nchunks, nvec, tab, n, col):
        ks = [int(np.floor(q * (n - 1))) for q in (0.25, 0.5, 0.75)]
        assert base % 8 == 0
        return [src, base // 8, nchunks, nvec, tab, ks[0], ks[1], ks[2], col, n]
    rows = [
        row(0, int(_EOFF[3]) * 25, 64, 800, 288, _WN[3], 6),
        row(0, int(_EOFF[1]) * 25, 64, 288, 0, _WN[1], 2),
        row(0, 0, 1, 216, 0, _WN[0], 0),
        row(1, 0 * 256, 1, _BN[0] // 16, 288, _BN[0], 1),
        row(1, 1 * 256, 1, _BN[1] // 16, 288, _BN[1], 3),
        row(1, 2 * 256, 1, _BN[2] // 16, 288, _BN[2], 5),
        row(1, 3 * 256, 1, _BN[3] // 16, 288, _BN[3], 7),
        row(0, int(_EOFF[2]) * 25, 128, 288, 0, _WN[2], 4),
    ]
    out = np.zeros((8, 16), np.int32)
    out[:, : len(rows[0])] = np.array(rows, np.int32)
    return out.reshape(-1)


def _recip_rows():
    t = np.zeros((8, 16), np.float32)
    ns = [_WN[3], _WN[0], _BN[0], _BN[1], _BN[2], _BN[3], _WN[2], _WN[1]]
    for i, n in enumerate(ns):
        t[i, 0] = np.float32(1.0 / n)
        t[i, 1] = np.float32(1.0 / (n - 1))
    return t.reshape(-1)


def _gidx_table():
    crop = (25 * np.arange(512)[:, None] + np.array(_CROP_F)[None, :]).reshape(-1)
    full = np.arange(_CPC)
    return np.concatenate([crop, full]).astype(np.int32)


def _key(xv):
    xi = plsc.bitcast(xv, jnp.int32)
    return xi ^ (lax.shift_right_arithmetic(xi, 31) | jnp.int32(-(2 ** 31)))


def _inv_key(k_scalar):
    kv = jnp.full((16,), k_scalar, jnp.int32)
    orig = jnp.where(kv < 0, kv & jnp.int32(0x7FFFFFFF), ~kv)
    return jnp.max(plsc.bitcast(orig, jnp.float32))


def _sc_body(xb_hbm, ef_hbm, gidx_hbm, pi_hbm, pf_hbm, out_hbm,
             buf, gidxv, piv, pfv, hist1, hist2, hist3, outrow):
    wid = lax.axis_index("s") * 2 + lax.axis_index("c")
    b = wid % _B
    g = wid // _B

    pltpu.sync_copy(gidx_hbm, gidxv)
    pltpu.sync_copy(pi_hbm, piv)
    pltpu.sync_copy(pf_hbm, pfv)

    ones = jnp.ones((16,), jnp.int32)
    zi = jnp.zeros((16,), jnp.int32)
    lanes = lax.iota(jnp.int32, 16)
    nunits = jnp.where(g == 0, 7, 1)

    def zero_hist(ref, n16):
        def zb(i, c):
            ref[pl.ds(i * 16, 16)] = zi
            return c
        lax.fori_loop(0, n16, zb, 0)

    def scan_hist(ref, base_off, n16, rank):
        def sb(i, carry):
            cum, bkt, resid = carry
            cs = plsc.cumsum(ref[pl.ds(base_off + i * 16, 16)])
            total = jnp.max(cs)
            cond = (cum + cs) > rank
            lane = jnp.sum(jnp.where(cond, 0, 1))
            cum_excl = cum + jnp.max(jnp.where(cond, 0, cs))
            found = (lane < 16) & (bkt < 0)
            bkt = jnp.where(found, i * 16 + lane, bkt)
            resid = jnp.where(found, rank - cum_excl, resid)
            return (cum + total, bkt, resid)
        _, bkt, resid = lax.fori_loop(0, n16, sb,
                                      (jnp.int32(0), jnp.int32(-1), jnp.int32(0)))
        return bkt, resid

    def unit_body(u, carry):
        urow = jnp.where(g == 0, u, 7 + u)
        pr = piv[pl.ds(urow * 16, 16)]
        prf = pfv[pl.ds(urow * 16, 16)]
        src = pr[0]
        base0 = pr[1]
        nchunks = pr[2]
        nvec = pr[3]
        tab = pr[4]
        col = pr[8]
        rn = prf[0]
        rn1 = prf[1]
        base8 = jnp.where(src == 0, b * (_EFLAT // 8), b * 128) + base0

        def stream(pass_fn, carry):
            def cb(ci, c):
                @pl.when(src == 0)
                def _():
                    pltpu.sync_copy(
                        ef_hbm.at[pl.ds((base8 + ci * (_CPC // 8)) * 8, _CPC)],
                        buf)

                @pl.when(src != 0)
                def _():
                    pltpu.sync_copy(xb_hbm.at[pl.ds(base8 * 8, 1024)],
                                    buf.at[pl.ds(0, 1024)])

                def vb(i, cc):
                    idx = gidxv[pl.ds((tab + i) * 16, 16)]
                    return pass_fn(plsc.load_gather(buf, [idx]), cc)
                return lax.fori_loop(0, nvec, vb, c)
            return lax.fori_loop(0, nchunks, cb, carry)

        zero_hist(hist1, 128)
        def p1(xv, c):
            s, s2, mn, mx = c
            plsc.addupdate_scatter(
                hist1, [lax.shift_right_logical(_key(xv), 21)], ones)
            return (s + xv, s2 + xv * xv, jnp.minimum(mn, xv),
                    jnp.maximum(mx, xv))
        big = jnp.float32(3.4e38)
        s, s2, mn, mx = stream(p1, (jnp.zeros((16,), jnp.float32),
                                    jnp.zeros((16,), jnp.float32),
                                    jnp.full((16,), big, jnp.float32),
                                    jnp.full((16,), -big, jnp.float32)))

        def search6(ref, n16, rank_of_t):
            def tb(t, c):
                bv, rv = c
                bkt, resid = scan_hist(ref, t * (n16 * 16), n16, rank_of_t(t))
                return (jnp.where(lanes == t, bkt, bv),
                        jnp.where(lanes == t, resid, rv))
            return lax.fori_loop(0, 6, tb, (zi, zi))

        def pick(vec, t):
            return jnp.max(jnp.where(lanes == t, vec, jnp.int32(-2 ** 31)))

        def rank1(t):
            idx = 5 + lax.shift_right_logical(t, 1)
            return jnp.max(jnp.where(lanes == idx, pr, jnp.int32(-2 ** 31))) \
                + (t & 1)

        def tscan1(t, c):
            bv, rv = c
            bkt, resid = scan_hist(hist1, 0, 128, rank1(t))
            return (jnp.where(lanes == t, bkt, bv),
                    jnp.where(lanes == t, resid, rv))
        b1v, r1v = lax.fori_loop(0, 6, tscan1, (zi, zi))
        b1s = [pick(b1v, t) for t in range(6)]

        zero_hist(hist2, 768)
        def p2(xv, c):
            k = _key(xv)
            bm = lax.shift_right_logical(k, 21)
            b2 = lax.shift_right_logical(k, 10) & jnp.int32(0x7FF)
            for t in range(6):
                plsc.addupdate_scatter(hist2, [t * 2048 + b2], ones,
                                       mask=bm == b1s[t])
            return c
        stream(p2, 0)
        b2v, r2v = search6(hist2, 128, lambda t: pick(r1v, t))
        b2s = [pick(b2v, t) for t in range(6)]

        zero_hist(hist3, 384)
        def p3(xv, c):
            k = _key(xv)
            bm = lax.shift_right_logical(k, 21)
            b2 = lax.shift_right_logical(k, 10) & jnp.int32(0x7FF)
            b3 = k & jnp.int32(0x3FF)
            for t in range(6):
                plsc.addupdate_scatter(hist3, [t * 1024 + b3], ones,
                                       mask=(bm == b1s[t]) & (b2 == b2s[t]))
            return c
        stream(p3, 0)
        b3v, _ = search6(hist3, 64, lambda t: pick(r2v, t))

        vals = []
        for t in range(6):
            kk = (lax.shift_left(b1s[t], 21) | lax.shift_left(b2s[t], 10)
                  | pick(b3v, t))
            vals.append(_inv_key(kk))

        S = jnp.sum(s)
        mean = S * rn
        var = (jnp.sum(s2) - S * S * rn) * rn1
        q25 = vals[0] * (np.float32(1) - _FR[0]) + vals[1] * _FR[0]
        q50 = vals[2] * (np.float32(1) - _FR[1]) + vals[3] * _FR[1]
        q75 = vals[4] * (np.float32(1) - _FR[2]) + vals[5] * _FR[2]
        ov = jnp.zeros((16,), jnp.float32)
        for slot, val in enumerate([mean, var, jnp.min(mn), q25, q50, q75,
                                    jnp.max(mx)]):
            ov = jnp.where(lanes == slot, val, ov)
        outrow[...] = ov
        pltpu.sync_copy(outrow, out_hbm.at[pl.ds((b * 8 + col) * 16, 16)])
        return carry

    lax.fori_loop(0, nunits, unit_body, 0)


@jax.jit
def _run(xb, ef, gidx, pi, pf):
    mesh = plsc.VectorSubcoreMesh(core_axis_name="c", subcore_axis_name="s")
    f = functools.partial(
        pl.kernel,
        mesh=mesh,
        compiler_params=pltpu.CompilerParams(needs_layout_passes=False),
        out_type=jax.ShapeDtypeStruct((2048,), jnp.float32),
        scratch_types=[
            pltpu.VMEM((_CPC,), jnp.float32),
            pltpu.VMEM((1088 * 16,), jnp.int32),
            pltpu.VMEM((128,), jnp.int32),
            pltpu.VMEM((128,), jnp.float32),
            pltpu.VMEM((2048,), jnp.int32),
            pltpu.VMEM((6 * 2048,), jnp.int32),
            pltpu.VMEM((6 * 1024,), jnp.int32),
            pltpu.VMEM((16,), jnp.float32),
        ],
    )(_sc_body)
    return f(xb, ef, gidx, pi, pf)


def kernel(x, edge_attr):
    segs = [jnp.pad(x[:, int(_NOFF[j + 1]):int(_NOFF[j + 2])],
                    ((0, 0), (0, 256 - _BN[j]))) for j in range(4)]
    xb = jnp.concatenate(segs, axis=1).reshape(-1)
    xb = jnp.pad(xb, (0, 1024))
    ef = edge_attr.reshape(-1)
    raw = _run(xb, ef, jnp.asarray(_gidx_table()), jnp.asarray(_unit_rows()),
               jnp.asarray(_recip_rows()))
    res = raw.reshape(_B, 8, 16)[:, :, :7].reshape(_B, 56)
    return jnp.pad(res, ((0, 0), (0, 14)))


# speedup vs baseline: 1.0855x; 1.0855x over previous
import functools

import numpy as np
import jax
import jax.numpy as jnp
from jax import lax
from jax.experimental import pallas as pl
from jax.experimental.pallas import tpu as pltpu
from jax.experimental.pallas import tpu_sc as plsc

_LAYOUT = [3, 128, 256, 256, 128]
_NOFF = np.cumsum([0] + _LAYOUT)
_ROWS = [_LAYOUT[j] * _LAYOUT[j + 1] for j in range(4)]
_EOFF = np.cumsum([0] + _ROWS)
_CROP_F = [6, 7, 8, 11, 12, 13, 16, 17, 18]
_B = 16
_EFLAT = int(_EOFF[4]) * 25
_CPC = 12800
_WN = [384 * 9, 32768 * 9, 65536 * 9, 32768 * 25]
_BN = [128, 256, 256, 128]
_FR = (np.float32(0.75), np.float32(0.5), np.float32(0.25))


def _unit_rows():
    def row(src, base, nchunks, nvec, tab, n, col):
        ks = [int(np.floor(q * (n - 1))) for q in (0.25, 0.5, 0.75)]
        assert base % 8 == 0
        return [src, base // 8, nchunks, nvec, tab, ks[0], ks[1], ks[2], col, n]
    rows = [
        row(0, int(_EOFF[3]) * 25, 64, 800, 288, _WN[3], 6),
        row(0, 0, 1, 216, 0, _WN[0], 0),
        row(1, 0 * 256, 1, _BN[0] // 16, 288, _BN[0], 1),
        row(1, 1 * 256, 1, _BN[1] // 16, 288, _BN[1], 3),
        row(1, 2 * 256, 1, _BN[2] // 16, 288, _BN[2], 5),
        row(1, 3 * 256, 1, _BN[3] // 16, 288, _BN[3], 7),
        row(0, int(_EOFF[2]) * 25, 128, 288, 0, _WN[2], 4),
        row(0, int(_EOFF[1]) * 25, 64, 288, 0, _WN[1], 2),
    ]
    out = np.zeros((8, 16), np.int32)
    out[:, : len(rows[0])] = np.array(rows, np.int32)
    return out.reshape(-1)


def _recip_rows():
    t = np.zeros((8, 16), np.float32)
    ns = [_WN[3], _WN[0], _BN[0], _BN[1], _BN[2], _BN[3], _WN[2], _WN[1]]
    for i, n in enumerate(ns):
        t[i, 0] = np.float32(1.0 / n)
        t[i, 1] = np.float32(1.0 / (n - 1))
    return t.reshape(-1)


def _gidx_table():
    crop = (25 * np.arange(512)[:, None] + np.array(_CROP_F)[None, :]).reshape(-1)
    full = np.arange(_CPC)
    return np.concatenate([crop, full]).astype(np.int32)


def _key(xv):
    xi = plsc.bitcast(xv, jnp.int32)
    return xi ^ (lax.shift_right_arithmetic(xi, 31) | jnp.int32(-(2 ** 31)))


def _inv_key(k_scalar):
    kv = jnp.full((16,), k_scalar, jnp.int32)
    orig = jnp.where(kv < 0, kv & jnp.int32(0x7FFFFFFF), ~kv)
    return jnp.max(plsc.bitcast(orig, jnp.float32))


def _sc_body(xb_hbm, ef_hbm, gidx_hbm, pi_hbm, pf_hbm, out_hbm,
             buf, gidxv, piv, pfv, hist1, hist2, hist3, outrow):
    wid = lax.axis_index("s") * 2 + lax.axis_index("c")
    b = wid % _B
    g = wid // _B

    pltpu.sync_copy(gidx_hbm, gidxv)
    pltpu.sync_copy(pi_hbm, piv)
    pltpu.sync_copy(pf_hbm, pfv)

    ones = jnp.ones((16,), jnp.int32)
    zi = jnp.zeros((16,), jnp.int32)
    lanes = lax.iota(jnp.int32, 16)
    nunits = jnp.where(g == 0, 6, 2)

    def zero_hist(ref, n16):
        def zb(i, c):
            ref[pl.ds(i * 16, 16)] = zi
            return c
        lax.fori_loop(0, n16, zb, 0)

    def scan_hist(ref, base_off, n16, rank):
        def sb(i, carry):
            cum, bkt, resid = carry
            cs = plsc.cumsum(ref[pl.ds(base_off + i * 16, 16)])
            total = jnp.max(cs)
            cond = (cum + cs) > rank
            lane = jnp.sum(jnp.where(cond, 0, 1))
            cum_excl = cum + jnp.max(jnp.where(cond, 0, cs))
            found = (lane < 16) & (bkt < 0)
            bkt = jnp.where(found, i * 16 + lane, bkt)
            resid = jnp.where(found, rank - cum_excl, resid)
            return (cum + total, bkt, resid)
        _, bkt, resid = lax.fori_loop(0, n16, sb,
                                      (jnp.int32(0), jnp.int32(-1), jnp.int32(0)))
        return bkt, resid

    def unit_body(u, carry):
        urow = jnp.where(g == 0, u, 6 + u)
        pr = piv[pl.ds(urow * 16, 16)]
        prf = pfv[pl.ds(urow * 16, 16)]
        src = pr[0]
        base0 = pr[1]
        nchunks = pr[2]
        nvec = pr[3]
        tab = pr[4]
        col = pr[8]
        rn = prf[0]
        rn1 = prf[1]
        base8 = jnp.where(src == 0, b * (_EFLAT // 8), b * 128) + base0

        def stream(pass_fn, carry):
            def cb(ci, c):
                @pl.when(src == 0)
                def _():
                    pltpu.sync_copy(
                        ef_hbm.at[pl.ds((base8 + ci * (_CPC // 8)) * 8, _CPC)],
                        buf)

                @pl.when(src != 0)
                def _():
                    pltpu.sync_copy(xb_hbm.at[pl.ds(base8 * 8, 1024)],
                                    buf.at[pl.ds(0, 1024)])

                def vb(i, cc):
                    idx = gidxv[pl.ds((tab + i) * 16, 16)]
                    return pass_fn(plsc.load_gather(buf, [idx]), cc)
                return lax.fori_loop(0, nvec, vb, c)
            return lax.fori_loop(0, nchunks, cb, carry)

        zero_hist(hist1, 128)
        def p1(xv, c):
            s, s2, mn, mx = c
            plsc.addupdate_scatter(
                hist1, [lax.shift_right_logical(_key(xv), 21)], ones)
            return (s + xv, s2 + xv * xv, jnp.minimum(mn, xv),
                    jnp.maximum(mx, xv))
        big = jnp.float32(3.4e38)
        s, s2, mn, mx = stream(p1, (jnp.zeros((16,), jnp.float32),
                                    jnp.zeros((16,), jnp.float32),
                                    jnp.full((16,), big, jnp.float32),
                                    jnp.full((16,), -big, jnp.float32)))

        def search6(ref, n16, rank_of_t, rows_v):
            def tb(t, c):
                bv, rv = c
                base = jnp.max(jnp.where(lanes == t, rows_v,
                                         jnp.int32(-2 ** 31))) * (n16 * 16)
                bkt, resid = scan_hist(ref, base, n16, rank_of_t(t))
                return (jnp.where(lanes == t, bkt, bv),
                        jnp.where(lanes == t, resid, rv))
            return lax.fori_loop(0, 6, tb, (zi, zi))

        def pick(vec, t):
            return jnp.max(jnp.where(lanes == t, vec, jnp.int32(-2 ** 31)))

        def rank1(t):
            idx = 5 + lax.shift_right_logical(t, 1)
            return jnp.max(jnp.where(lanes == idx, pr, jnp.int32(-2 ** 31))) \
                + (t & 1)

        def tscan1(t, c):
            bv, rv = c
            bkt, resid = scan_hist(hist1, 0, 128, rank1(t))
            return (jnp.where(lanes == t, bkt, bv),
                    jnp.where(lanes == t, resid, rv))
        b1v, r1v = lax.fori_loop(0, 6, tscan1, (zi, zi))
        b1s = [pick(b1v, t) for t in range(6)]

        m2v = zi
        for t in range(6):
            mt = jnp.int32(t)
            for tp in reversed(range(t)):
                mt = jnp.where(b1s[tp] == b1s[t], jnp.int32(tp), mt)
            m2v = jnp.where(lanes == t, mt, m2v)

        dump2 = jnp.int32(6 * 2048) + lanes
        dump3 = jnp.int32(6 * 1024) + lanes

        zero_hist(hist2, 768)
        def p2(xv, c):
            k = _key(xv)
            bm = lax.shift_right_logical(k, 21)
            b2 = lax.shift_right_logical(k, 10) & jnp.int32(0x7FF)
            addr = dump2
            for t in reversed(range(6)):
                addr = jnp.where(bm == b1s[t], t * 2048 + b2, addr)
            plsc.addupdate_scatter(hist2, [addr], ones)
            return c
        stream(p2, 0)
        b2v, r2v = search6(hist2, 128, lambda t: pick(r1v, t), m2v)
        b2s = [pick(b2v, t) for t in range(6)]

        m3v = zi
        for t in range(6):
            mt = jnp.int32(t)
            for tp in reversed(range(t)):
                same = (b1s[tp] == b1s[t]) & (b2s[tp] == b2s[t])
                mt = jnp.where(same, jnp.int32(tp), mt)
            m3v = jnp.where(lanes == t, mt, m3v)

        zero_hist(hist3, 384)
        def p3(xv, c):
            k = _key(xv)
            bm = lax.shift_right_logical(k, 21)
            b2 = lax.shift_right_logical(k, 10) & jnp.int32(0x7FF)
            b3 = k & jnp.int32(0x3FF)
            addr = dump3
            for t in reversed(range(6)):
                addr = jnp.where((bm == b1s[t]) & (b2 == b2s[t]),
                                 t * 1024 + b3, addr)
            plsc.addupdate_scatter(hist3, [addr], ones)
            return c
        stream(p3, 0)
        b3v, _ = search6(hist3, 64, lambda t: pick(r2v, t), m3v)

        vals = []
        for t in range(6):
            kk = (lax.shift_left(b1s[t], 21) | lax.shift_left(b2s[t], 10)
                  | pick(b3v, t))
            vals.append(_inv_key(kk))

        S = jnp.sum(s)
        mean = S * rn
        var = (jnp.sum(s2) - S * S * rn) * rn1
        q25 = vals[0] * (np.float32(1) - _FR[0]) + vals[1] * _FR[0]
        q50 = vals[2] * (np.float32(1) - _FR[1]) + vals[3] * _FR[1]
        q75 = vals[4] * (np.float32(1) - _FR[2]) + vals[5] * _FR[2]
        ov = jnp.zeros((16,), jnp.float32)
        for slot, val in enumerate([mean, var, jnp.min(mn), q25, q50, q75,
                                    jnp.max(mx)]):
            ov = jnp.where(lanes == slot, val, ov)
        outrow[...] = ov
        pltpu.sync_copy(outrow, out_hbm.at[pl.ds((b * 8 + col) * 16, 16)])
        return carry

    lax.fori_loop(0, nunits, unit_body, 0)


@jax.jit
def _run(xb, ef, gidx, pi, pf):
    mesh = plsc.VectorSubcoreMesh(core_axis_name="c", subcore_axis_name="s")
    f = functools.partial(
        pl.kernel,
        mesh=mesh,
        compiler_params=pltpu.CompilerParams(needs_layout_passes=False),
        out_type=jax.ShapeDtypeStruct((2048,), jnp.float32),
        scratch_types=[
            pltpu.VMEM((_CPC,), jnp.float32),
            pltpu.VMEM((1088 * 16,), jnp.int32),
            pltpu.VMEM((128,), jnp.int32),
            pltpu.VMEM((128,), jnp.float32),
            pltpu.VMEM((2048,), jnp.int32),
            pltpu.VMEM((6 * 2048 + 16,), jnp.int32),
            pltpu.VMEM((6 * 1024 + 16,), jnp.int32),
            pltpu.VMEM((16,), jnp.float32),
        ],
    )(_sc_body)
    return f(xb, ef, gidx, pi, pf)


def kernel(x, edge_attr):
    segs = [jnp.pad(x[:, int(_NOFF[j + 1]):int(_NOFF[j + 2])],
                    ((0, 0), (0, 256 - _BN[j]))) for j in range(4)]
    xb = jnp.concatenate(segs, axis=1).reshape(-1)
    xb = jnp.pad(xb, (0, 1024))
    ef = edge_attr.reshape(-1)
    raw = _run(xb, ef, jnp.asarray(_gidx_table()), jnp.asarray(_unit_rows()),
               jnp.asarray(_recip_rows()))
    res = raw.reshape(_B, 8, 16)[:, :, :7].reshape(_B, 56)
    return jnp.pad(res, ((0, 0), (0, 14)))


# speedup vs baseline: 1.2582x; 1.1591x over previous
import functools

import numpy as np
import jax
import jax.numpy as jnp
from jax import lax
from jax.experimental import pallas as pl
from jax.experimental.pallas import tpu as pltpu
from jax.experimental.pallas import tpu_sc as plsc

_LAYOUT = [3, 128, 256, 256, 128]
_NOFF = np.cumsum([0] + _LAYOUT)
_ROWS = [_LAYOUT[j] * _LAYOUT[j + 1] for j in range(4)]
_EOFF = np.cumsum([0] + _ROWS)
_CROP_F = [6, 7, 8, 11, 12, 13, 16, 17, 18]
_B = 16
_EFLAT = int(_EOFF[4]) * 25
_CPC = 12800
_WN = [384 * 9, 32768 * 9, 65536 * 9, 32768 * 25]
_BN = [128, 256, 256, 128]
_FR = (np.float32(0.75), np.float32(0.5), np.float32(0.25))


def _unit_rows():
    def row(src, base, nchunks, nvec, tab, n, col):
        ks = [int(np.floor(q * (n - 1))) for q in (0.25, 0.5, 0.75)]
        assert base % 8 == 0
        return [src, base // 8, nchunks, nvec, tab, ks[0], ks[1], ks[2], col, n]
    rows = [
        row(0, int(_EOFF[3]) * 25, 64, 800, 288, _WN[3], 6),
        row(0, 0, 1, 216, 0, _WN[0], 0),
        row(1, 0 * 256, 1, _BN[0] // 16, 288, _BN[0], 1),
        row(1, 1 * 256, 1, _BN[1] // 16, 288, _BN[1], 3),
        row(1, 2 * 256, 1, _BN[2] // 16, 288, _BN[2], 5),
        row(1, 3 * 256, 1, _BN[3] // 16, 288, _BN[3], 7),
        row(0, int(_EOFF[2]) * 25, 128, 288, 0, _WN[2], 4),
        row(0, int(_EOFF[1]) * 25, 64, 288, 0, _WN[1], 2),
    ]
    out = np.zeros((8, 16), np.int32)
    out[:, : len(rows[0])] = np.array(rows, np.int32)
    return out.reshape(-1)


def _recip_rows():
    t = np.zeros((8, 16), np.float32)
    ns = [_WN[3], _WN[0], _BN[0], _BN[1], _BN[2], _BN[3], _WN[2], _WN[1]]
    for i, n in enumerate(ns):
        t[i, 0] = np.float32(1.0 / n)
        t[i, 1] = np.float32(1.0 / (n - 1))
    return t.reshape(-1)


def _gidx_table():
    crop = (25 * np.arange(512)[:, None] + np.array(_CROP_F)[None, :]).reshape(-1)
    full = np.arange(_CPC)
    return np.concatenate([crop, full]).astype(np.int32)


def _key(xv):
    xi = plsc.bitcast(xv, jnp.int32)
    return xi ^ (lax.shift_right_arithmetic(xi, 31) | jnp.int32(-(2 ** 31)))


def _inv_key(k_scalar):
    kv = jnp.full((16,), k_scalar, jnp.int32)
    orig = jnp.where(kv < 0, kv & jnp.int32(0x7FFFFFFF), ~kv)
    return jnp.max(plsc.bitcast(orig, jnp.float32))


def _sc_body(xb_hbm, ef_hbm, gidx_hbm, pi_hbm, pf_hbm, out_hbm,
             buf, gidxv, piv, pfv, hist1, hist2, hist3, outrow, sem):
    wid = lax.axis_index("s") * 2 + lax.axis_index("c")
    b = wid % _B
    g = wid // _B

    pltpu.sync_copy(gidx_hbm, gidxv)
    pltpu.sync_copy(pi_hbm, piv)
    pltpu.sync_copy(pf_hbm, pfv)

    ones = jnp.ones((16,), jnp.int32)
    zi = jnp.zeros((16,), jnp.int32)
    lanes = lax.iota(jnp.int32, 16)
    nunits = jnp.where(g == 0, 6, 2)

    def zero_hist(ref, n16):
        def zb(i, c):
            ref[pl.ds(i * 16, 16)] = zi
            return c
        lax.fori_loop(0, n16, zb, 0)

    def scan_hist(ref, base_off, n16, rank):
        def sb(i, carry):
            cum, bkt, resid = carry
            cs = plsc.cumsum(ref[pl.ds(base_off + i * 16, 16)])
            total = jnp.max(cs)
            cond = (cum + cs) > rank
            lane = jnp.sum(jnp.where(cond, 0, 1))
            cum_excl = cum + jnp.max(jnp.where(cond, 0, cs))
            found = (lane < 16) & (bkt < 0)
            bkt = jnp.where(found, i * 16 + lane, bkt)
            resid = jnp.where(found, rank - cum_excl, resid)
            return (cum + total, bkt, resid)
        _, bkt, resid = lax.fori_loop(0, n16, sb,
                                      (jnp.int32(0), jnp.int32(-1), jnp.int32(0)))
        return bkt, resid

    def unit_body(u, carry):
        urow = jnp.where(g == 0, u, 6 + u)
        pr = piv[pl.ds(urow * 16, 16)]
        prf = pfv[pl.ds(urow * 16, 16)]
        src = pr[0]
        base0 = pr[1]
        nchunks = pr[2]
        nvec = pr[3]
        tab = pr[4]
        col = pr[8]
        rn = prf[0]
        rn1 = prf[1]
        base8 = jnp.where(src == 0, b * (_EFLAT // 8), b * 128) + base0

        def stream(pass_fn, carry):
            @pl.when(src == 0)
            def _():
                pltpu.sync_copy(ef_hbm.at[pl.ds(base8 * 8, _CPC)],
                                buf.at[pl.ds(0, _CPC)])

            @pl.when(src != 0)
            def _():
                pltpu.sync_copy(xb_hbm.at[pl.ds(base8 * 8, 1024)],
                                buf.at[pl.ds(0, 1024)])

            def cb(ci, c):
                @pl.when(ci + 1 < nchunks)
                def _():
                    off8 = (base8 + (ci + 1) * (_CPC // 8)) * 8
                    half = ((ci + 1) & 1) * _CPC
                    pltpu.async_copy(ef_hbm.at[pl.ds(off8, _CPC)],
                                     buf.at[pl.ds(half, _CPC)], sem)

                boff = (ci & 1) * _CPC

                def vb(i, cc):
                    idx = gidxv[pl.ds((tab + i) * 16, 16)] + boff
                    return pass_fn(plsc.load_gather(buf, [idx]), cc)
                c2 = lax.fori_loop(0, nvec, vb, c)

                @pl.when(ci + 1 < nchunks)
                def _():
                    off8 = (base8 + (ci + 1) * (_CPC // 8)) * 8
                    half = ((ci + 1) & 1) * _CPC
                    pltpu.make_async_copy(ef_hbm.at[pl.ds(off8, _CPC)],
                                          buf.at[pl.ds(half, _CPC)],
                                          sem).wait()
                return c2
            return lax.fori_loop(0, nchunks, cb, carry)

        zero_hist(hist1, 128)
        def p1(xv, c):
            s, s2, mn, mx = c
            plsc.addupdate_scatter(
                hist1, [lax.shift_right_logical(_key(xv), 21)], ones)
            return (s + xv, s2 + xv * xv, jnp.minimum(mn, xv),
                    jnp.maximum(mx, xv))
        big = jnp.float32(3.4e38)
        s, s2, mn, mx = stream(p1, (jnp.zeros((16,), jnp.float32),
                                    jnp.zeros((16,), jnp.float32),
                                    jnp.full((16,), big, jnp.float32),
                                    jnp.full((16,), -big, jnp.float32)))

        def search6(ref, n16, rank_of_t, rows_v):
            def tb(t, c):
                bv, rv = c
                base = jnp.max(jnp.where(lanes == t, rows_v,
                                         jnp.int32(-2 ** 31))) * (n16 * 16)
                bkt, resid = scan_hist(ref, base, n16, rank_of_t(t))
                return (jnp.where(lanes == t, bkt, bv),
                        jnp.where(lanes == t, resid, rv))
            return lax.fori_loop(0, 6, tb, (zi, zi))

        def pick(vec, t):
            return jnp.max(jnp.where(lanes == t, vec, jnp.int32(-2 ** 31)))

        def rank1(t):
            idx = 5 + lax.shift_right_logical(t, 1)
            return jnp.max(jnp.where(lanes == idx, pr, jnp.int32(-2 ** 31))) \
                + (t & 1)

        def tscan1(t, c):
            bv, rv = c
            bkt, resid = scan_hist(hist1, 0, 128, rank1(t))
            return (jnp.where(lanes == t, bkt, bv),
                    jnp.where(lanes == t, resid, rv))
        b1v, r1v = lax.fori_loop(0, 6, tscan1, (zi, zi))
        b1s = [pick(b1v, t) for t in range(6)]

        zero_hist(hist2, 768)
        def p2(xv, c):
            k = _key(xv)
            bm = lax.shift_right_logical(k, 21)
            b2 = lax.shift_right_logical(k, 10) & jnp.int32(0x7FF)
            for t in range(6):
                plsc.addupdate_scatter(hist2, [t * 2048 + b2], ones,
                                       mask=bm == b1s[t])
            return c
        stream(p2, 0)
        b2v, r2v = search6(hist2, 128, lambda t: pick(r1v, t), lanes)
        b2s = [pick(b2v, t) for t in range(6)]

        zero_hist(hist3, 384)
        def p3(xv, c):
            k = _key(xv)
            bm = lax.shift_right_logical(k, 21)
            b2 = lax.shift_right_logical(k, 10) & jnp.int32(0x7FF)
            b3 = k & jnp.int32(0x3FF)
            for t in range(6):
                plsc.addupdate_scatter(hist3, [t * 1024 + b3], ones,
                                       mask=(bm == b1s[t]) & (b2 == b2s[t]))
            return c
        stream(p3, 0)
        b3v, _ = search6(hist3, 64, lambda t: pick(r2v, t), lanes)

        vals = []
        for t in range(6):
            kk = (lax.shift_left(b1s[t], 21) | lax.shift_left(b2s[t], 10)
                  | pick(b3v, t))
            vals.append(_inv_key(kk))

        S = jnp.sum(s)
        mean = S * rn
        var = (jnp.sum(s2) - S * S * rn) * rn1
        q25 = vals[0] * (np.float32(1) - _FR[0]) + vals[1] * _FR[0]
        q50 = vals[2] * (np.float32(1) - _FR[1]) + vals[3] * _FR[1]
        q75 = vals[4] * (np.float32(1) - _FR[2]) + vals[5] * _FR[2]
        ov = jnp.zeros((16,), jnp.float32)
        for slot, val in enumerate([mean, var, jnp.min(mn), q25, q50, q75,
                                    jnp.max(mx)]):
            ov = jnp.where(lanes == slot, val, ov)
        outrow[...] = ov
        pltpu.sync_copy(outrow, out_hbm.at[pl.ds((b * 8 + col) * 16, 16)])
        return carry

    lax.fori_loop(0, nunits, unit_body, 0)


@jax.jit
def _run(xb, ef, gidx, pi, pf):
    mesh = plsc.VectorSubcoreMesh(core_axis_name="c", subcore_axis_name="s")
    f = functools.partial(
        pl.kernel,
        mesh=mesh,
        compiler_params=pltpu.CompilerParams(needs_layout_passes=False),
        out_type=jax.ShapeDtypeStruct((2048,), jnp.float32),
        scratch_types=[
            pltpu.VMEM((2 * _CPC,), jnp.float32),
            pltpu.VMEM((1088 * 16,), jnp.int32),
            pltpu.VMEM((128,), jnp.int32),
            pltpu.VMEM((128,), jnp.float32),
            pltpu.VMEM((2048,), jnp.int32),
            pltpu.VMEM((6 * 2048 + 16,), jnp.int32),
            pltpu.VMEM((6 * 1024 + 16,), jnp.int32),
            pltpu.VMEM((16,), jnp.float32),
            pltpu.SemaphoreType.DMA,
        ],
    )(_sc_body)
    return f(xb, ef, gidx, pi, pf)


def kernel(x, edge_attr):
    segs = [jnp.pad(x[:, int(_NOFF[j + 1]):int(_NOFF[j + 2])],
                    ((0, 0), (0, 256 - _BN[j]))) for j in range(4)]
    xb = jnp.concatenate(segs, axis=1).reshape(-1)
    xb = jnp.pad(xb, (0, 1024))
    ef = edge_attr.reshape(-1)
    raw = _run(xb, ef, jnp.asarray(_gidx_table()), jnp.asarray(_unit_rows()),
               jnp.asarray(_recip_rows()))
    res = raw.reshape(_B, 8, 16)[:, :, :7].reshape(_B, 56)
    return jnp.pad(res, ((0, 0), (0, 14)))


# speedup vs baseline: 1.4380x; 1.1429x over previous
import functools

import numpy as np
import jax
import jax.numpy as jnp
from jax import lax
from jax.experimental import pallas as pl
from jax.experimental.pallas import tpu as pltpu
from jax.experimental.pallas import tpu_sc as plsc

_LAYOUT = [3, 128, 256, 256, 128]
_NOFF = np.cumsum([0] + _LAYOUT)
_ROWS = [_LAYOUT[j] * _LAYOUT[j + 1] for j in range(4)]
_EOFF = np.cumsum([0] + _ROWS)
_CROP_F = [6, 7, 8, 11, 12, 13, 16, 17, 18]
_B = 16
_EFLAT = int(_EOFF[4]) * 25
_CPC = 12800
_WN = [384 * 9, 32768 * 9, 65536 * 9, 32768 * 25]
_BN = [128, 256, 256, 128]
_FR = (np.float32(0.75), np.float32(0.5), np.float32(0.25))


def _unit_rows():
    def row(src, base, nchunks, nvec, tab, n, col):
        ks = [int(np.floor(q * (n - 1))) for q in (0.25, 0.5, 0.75)]
        assert base % 8 == 0
        return [src, base // 8, nchunks, nvec, tab, ks[0], ks[1], ks[2], col, n]
    rows = [
        row(0, int(_EOFF[3]) * 25, 64, 800, 288, _WN[3], 6),
        row(0, 0, 1, 216, 0, _WN[0], 0),
        row(1, 0 * 256, 1, _BN[0] // 16, 288, _BN[0], 1),
        row(1, 1 * 256, 1, _BN[1] // 16, 288, _BN[1], 3),
        row(1, 2 * 256, 1, _BN[2] // 16, 288, _BN[2], 5),
        row(1, 3 * 256, 1, _BN[3] // 16, 288, _BN[3], 7),
        row(0, int(_EOFF[2]) * 25, 128, 288, 0, _WN[2], 4),
        row(0, int(_EOFF[1]) * 25, 64, 288, 0, _WN[1], 2),
    ]
    out = np.zeros((8, 16), np.int32)
    out[:, : len(rows[0])] = np.array(rows, np.int32)
    return out.reshape(-1)


def _recip_rows():
    t = np.zeros((8, 16), np.float32)
    ns = [_WN[3], _WN[0], _BN[0], _BN[1], _BN[2], _BN[3], _WN[2], _WN[1]]
    for i, n in enumerate(ns):
        t[i, 0] = np.float32(1.0 / n)
        t[i, 1] = np.float32(1.0 / (n - 1))
    return t.reshape(-1)


def _gidx_table():
    crop = (25 * np.arange(512)[:, None] + np.array(_CROP_F)[None, :]).reshape(-1)
    full = np.arange(_CPC)
    return np.concatenate([crop, full]).astype(np.int32)


def _key(xv):
    xi = plsc.bitcast(xv, jnp.int32)
    return xi ^ (lax.shift_right_arithmetic(xi, 31) | jnp.int32(-(2 ** 31)))


def _inv_key(k_scalar):
    kv = jnp.full((16,), k_scalar, jnp.int32)
    orig = jnp.where(kv < 0, kv & jnp.int32(0x7FFFFFFF), ~kv)
    return jnp.max(plsc.bitcast(orig, jnp.float32))


def _sc_body(xb_hbm, ef_hbm, gidx_hbm, pi_hbm, pf_hbm, out_hbm,
             buf, gidxv, piv, pfv, hist1, hist2, hist3, outrow, sem):
    wid = lax.axis_index("s") * 2 + lax.axis_index("c")
    b = wid % _B
    g = wid // _B

    pltpu.sync_copy(gidx_hbm, gidxv)
    pltpu.sync_copy(pi_hbm, piv)
    pltpu.sync_copy(pf_hbm, pfv)

    ones = jnp.ones((16,), jnp.int32)
    zi = jnp.zeros((16,), jnp.int32)
    lanes = lax.iota(jnp.int32, 16)
    nunits = jnp.where(g == 0, 6, 2)

    def zero_hist(ref, n16):
        def zb(i, c):
            ref[pl.ds(i * 16, 16)] = zi
            return c
        lax.fori_loop(0, n16, zb, 0)

    def scan_hist(ref, base_off, n16, rank):
        def sb(i, carry):
            cum, bkt, resid = carry
            cs = plsc.cumsum(ref[pl.ds(base_off + i * 16, 16)])
            total = jnp.max(cs)
            cond = (cum + cs) > rank
            lane = jnp.sum(jnp.where(cond, 0, 1))
            cum_excl = cum + jnp.max(jnp.where(cond, 0, cs))
            found = (lane < 16) & (bkt < 0)
            bkt = jnp.where(found, i * 16 + lane, bkt)
            resid = jnp.where(found, rank - cum_excl, resid)
            return (cum + total, bkt, resid)
        _, bkt, resid = lax.fori_loop(0, n16, sb,
                                      (jnp.int32(0), jnp.int32(-1), jnp.int32(0)))
        return bkt, resid

    def unit_body(u, carry):
        urow = jnp.where(g == 0, u, 6 + u)
        pr = piv[pl.ds(urow * 16, 16)]
        prf = pfv[pl.ds(urow * 16, 16)]
        src = pr[0]
        base0 = pr[1]
        nchunks = pr[2]
        nvec = pr[3]
        tab = pr[4]
        col = pr[8]
        rn = prf[0]
        rn1 = prf[1]
        base8 = jnp.where(src == 0, b * (_EFLAT // 8), b * 128) + base0

        def stream(pass_fn, carry):
            @pl.when(src == 0)
            def _():
                pltpu.sync_copy(ef_hbm.at[pl.ds(base8 * 8, _CPC)],
                                buf.at[pl.ds(0, _CPC)])

            @pl.when(src != 0)
            def _():
                pltpu.sync_copy(xb_hbm.at[pl.ds(base8 * 8, 1024)],
                                buf.at[pl.ds(0, 1024)])

            def cb(ci, c):
                @pl.when(ci + 1 < nchunks)
                def _():
                    off8 = (base8 + (ci + 1) * (_CPC // 8)) * 8
                    half = ((ci + 1) & 1) * _CPC
                    pltpu.async_copy(ef_hbm.at[pl.ds(off8, _CPC)],
                                     buf.at[pl.ds(half, _CPC)], sem)

                boff = (ci & 1) * _CPC

                def vb(i, cc):
                    row = (tab + i * 2) * 16
                    idx0 = gidxv[pl.ds(row, 16)] + boff
                    idx1 = gidxv[pl.ds(row + 16, 16)] + boff
                    cc = pass_fn(plsc.load_gather(buf, [idx0]), cc)
                    return pass_fn(plsc.load_gather(buf, [idx1]), cc)
                c2 = lax.fori_loop(0, lax.shift_right_logical(nvec, 1), vb, c)

                @pl.when(ci + 1 < nchunks)
                def _():
                    off8 = (base8 + (ci + 1) * (_CPC // 8)) * 8
                    half = ((ci + 1) & 1) * _CPC
                    pltpu.make_async_copy(ef_hbm.at[pl.ds(off8, _CPC)],
                                          buf.at[pl.ds(half, _CPC)],
                                          sem).wait()
                return c2
            return lax.fori_loop(0, nchunks, cb, carry)

        zero_hist(hist1, 128)
        def p1(xv, c):
            s, s2, mn, mx = c
            plsc.addupdate_scatter(
                hist1, [lax.shift_right_logical(_key(xv), 21)], ones)
            return (s + xv, s2 + xv * xv, jnp.minimum(mn, xv),
                    jnp.maximum(mx, xv))
        big = jnp.float32(3.4e38)
        s, s2, mn, mx = stream(p1, (jnp.zeros((16,), jnp.float32),
                                    jnp.zeros((16,), jnp.float32),
                                    jnp.full((16,), big, jnp.float32),
                                    jnp.full((16,), -big, jnp.float32)))

        def search6(ref, n16, rank_of_t, rows_v):
            def tb(t, c):
                bv, rv = c
                base = jnp.max(jnp.where(lanes == t, rows_v,
                                         jnp.int32(-2 ** 31))) * (n16 * 16)
                bkt, resid = scan_hist(ref, base, n16, rank_of_t(t))
                return (jnp.where(lanes == t, bkt, bv),
                        jnp.where(lanes == t, resid, rv))
            return lax.fori_loop(0, 6, tb, (zi, zi))

        def pick(vec, t):
            return jnp.max(jnp.where(lanes == t, vec, jnp.int32(-2 ** 31)))

        def rank1(t):
            idx = 5 + lax.shift_right_logical(t, 1)
            return jnp.max(jnp.where(lanes == idx, pr, jnp.int32(-2 ** 31))) \
                + (t & 1)

        def tscan1(t, c):
            bv, rv = c
            bkt, resid = scan_hist(hist1, 0, 128, rank1(t))
            return (jnp.where(lanes == t, bkt, bv),
                    jnp.where(lanes == t, resid, rv))
        b1v, r1v = lax.fori_loop(0, 6, tscan1, (zi, zi))
        b1s = [pick(b1v, t) for t in range(6)]

        zero_hist(hist2, 768)
        def p2(xv, c):
            k = _key(xv)
            bm = lax.shift_right_logical(k, 21)
            b2 = lax.shift_right_logical(k, 10) & jnp.int32(0x7FF)
            for t in range(6):
                plsc.addupdate_scatter(hist2, [t * 2048 + b2], ones,
                                       mask=bm == b1s[t])
            return c
        stream(p2, 0)
        b2v, r2v = search6(hist2, 128, lambda t: pick(r1v, t), lanes)
        b2s = [pick(b2v, t) for t in range(6)]

        zero_hist(hist3, 384)
        def p3(xv, c):
            k = _key(xv)
            bm = lax.shift_right_logical(k, 21)
            b2 = lax.shift_right_logical(k, 10) & jnp.int32(0x7FF)
            b3 = k & jnp.int32(0x3FF)
            for t in range(6):
                plsc.addupdate_scatter(hist3, [t * 1024 + b3], ones,
                                       mask=(bm == b1s[t]) & (b2 == b2s[t]))
            return c
        stream(p3, 0)
        b3v, _ = search6(hist3, 64, lambda t: pick(r2v, t), lanes)

        vals = []
        for t in range(6):
            kk = (lax.shift_left(b1s[t], 21) | lax.shift_left(b2s[t], 10)
                  | pick(b3v, t))
            vals.append(_inv_key(kk))

        S = jnp.sum(s)
        mean = S * rn
        var = (jnp.sum(s2) - S * S * rn) * rn1
        q25 = vals[0] * (np.float32(1) - _FR[0]) + vals[1] * _FR[0]
        q50 = vals[2] * (np.float32(1) - _FR[1]) + vals[3] * _FR[1]
        q75 = vals[4] * (np.float32(1) - _FR[2]) + vals[5] * _FR[2]
        ov = jnp.zeros((16,), jnp.float32)
        for slot, val in enumerate([mean, var, jnp.min(mn), q25, q50, q75,
                                    jnp.max(mx)]):
            ov = jnp.where(lanes == slot, val, ov)
        outrow[...] = ov
        pltpu.sync_copy(outrow, out_hbm.at[pl.ds((b * 8 + col) * 16, 16)])
        return carry

    lax.fori_loop(0, nunits, unit_body, 0)


@jax.jit
def _run(xb, ef, gidx, pi, pf):
    mesh = plsc.VectorSubcoreMesh(core_axis_name="c", subcore_axis_name="s")
    f = functools.partial(
        pl.kernel,
        mesh=mesh,
        compiler_params=pltpu.CompilerParams(needs_layout_passes=False),
        out_type=jax.ShapeDtypeStruct((2048,), jnp.float32),
        scratch_types=[
            pltpu.VMEM((2 * _CPC,), jnp.float32),
            pltpu.VMEM((1088 * 16,), jnp.int32),
            pltpu.VMEM((128,), jnp.int32),
            pltpu.VMEM((128,), jnp.float32),
            pltpu.VMEM((2048,), jnp.int32),
            pltpu.VMEM((6 * 2048 + 16,), jnp.int32),
            pltpu.VMEM((6 * 1024 + 16,), jnp.int32),
            pltpu.VMEM((16,), jnp.float32),
            pltpu.SemaphoreType.DMA,
        ],
    )(_sc_body)
    return f(xb, ef, gidx, pi, pf)


def kernel(x, edge_attr):
    segs = [jnp.pad(x[:, int(_NOFF[j + 1]):int(_NOFF[j + 2])],
                    ((0, 0), (0, 256 - _BN[j]))) for j in range(4)]
    xb = jnp.concatenate(segs, axis=1).reshape(-1)
    xb = jnp.pad(xb, (0, 1024))
    ef = edge_attr.reshape(-1)
    raw = _run(xb, ef, jnp.asarray(_gidx_table()), jnp.asarray(_unit_rows()),
               jnp.asarray(_recip_rows()))
    res = raw.reshape(_B, 8, 16)[:, :, :7].reshape(_B, 56)
    return jnp.pad(res, ((0, 0), (0, 14)))
